# baseline (speedup 1.0000x reference)
import functools

import jax
import jax.numpy as jnp
from jax import lax
from jax.experimental import pallas as pl
from jax.experimental.pallas import tpu as pltpu
from jax.experimental.pallas import tpu_sc as plsc

NC = 2
NS = 16
NW = NC * NS

NBUF = 8


def _make_gather(batch: int, s_len: int, dim: int):
    assert batch % NW == 0
    rows_per_w = batch // NW
    c0 = min(128, (s_len + 1) // 2 + (-((s_len + 1) // 2) % 8))
    c1 = s_len - c0
    assert 0 < c1 <= 128 and c0 % 8 == 0
    chunks_per_row = 2
    n_chunks = rows_per_w * chunks_per_row
    assert n_chunks % NBUF == 0
    n_groups = n_chunks // NBUF

    mesh = plsc.VectorSubcoreMesh(
        core_axis_name="c", subcore_axis_name="s",
        num_cores=NC, num_subcores=NS)

    def chunk_params(j):
        p = j % 2
        off = 0 if p == 0 else c0
        ln = c0 if p == 0 else c1
        return off, ln

    @functools.partial(
        pl.kernel,
        out_type=jax.ShapeDtypeStruct((batch, s_len, dim), jnp.float32),
        mesh=mesh,
        scratch_types=[
            pltpu.VMEM((rows_per_w, s_len), jnp.int32),
            pltpu.VMEM((NBUF, c0, dim), jnp.float32),
            pltpu.SemaphoreType.DMA((NBUF,)),
            pltpu.SemaphoreType.DMA((NBUF,)),
        ],
        compiler_params=pltpu.CompilerParams(use_tc_tiling_on_sc=False),
    )
    def gather_kernel(vocab_hbm, s_hbm, out_hbm, idx_v, rows_v, gsem, wsem):
        wid = lax.axis_index("s") * NC + lax.axis_index("c")
        b0 = wid * rows_per_w
        pltpu.sync_copy(s_hbm.at[pl.ds(b0, rows_per_w)], idx_v)

        def start_gather(k, j):
            off, ln = chunk_params(j)
            r = k // chunks_per_row
            pltpu.async_copy(
                vocab_hbm.at[idx_v.at[r, pl.ds(off, ln)]],
                rows_v.at[j, pl.ds(0, ln)], gsem.at[j])

        def start_write(k, j):
            off, ln = chunk_params(j)
            r = k // chunks_per_row
            pltpu.async_copy(
                rows_v.at[j, pl.ds(0, ln)],
                out_hbm.at[b0 + r, pl.ds(off, ln)], wsem.at[j])

        def wait_gather(j):
            _, ln = chunk_params(j)
            pltpu.make_async_copy(
                vocab_hbm.at[idx_v.at[0, pl.ds(0, ln)]],
                rows_v.at[j, pl.ds(0, ln)], gsem.at[j]).wait()

        def wait_write(j):
            off, ln = chunk_params(j)
            pltpu.make_async_copy(
                rows_v.at[j, pl.ds(0, ln)],
                out_hbm.at[b0, pl.ds(off, ln)], wsem.at[j]).wait()

        for j in range(NBUF):
            start_gather(j, j)

        def body(g, carry):
            for j in range(NBUF):
                wait_gather(j)
                start_write(g * NBUF + j, j)
            for j in range(NBUF):
                wait_write(j)
                start_gather((g + 1) * NBUF + j, j)
            return carry

        lax.fori_loop(0, n_groups - 1, body, 0, unroll=False)

        last = (n_groups - 1) * NBUF
        for j in range(NBUF):
            wait_gather(j)
            start_write(last + j, j)
        for j in range(NBUF):
            wait_write(j)

    return gather_kernel


def kernel(s, vocab):
    b, t = s.shape
    dim = vocab.shape[1]
    return _make_gather(b, t, dim)(vocab, s.astype(jnp.int32))


# speedup vs baseline: 1.2196x; 1.2196x over previous
import functools

import jax
import jax.numpy as jnp
from jax import lax
from jax.experimental import pallas as pl
from jax.experimental.pallas import tpu as pltpu
from jax.experimental.pallas import tpu_sc as plsc

NC = 2
NS = 16
NW = NC * NS

NBUF = 5
CHUNK = 128
PADW = 128


def _make_gather(n_total: int):
    assert n_total % (NW * CHUNK) == 0
    per_w = n_total // NW
    n_chunks = per_w // CHUNK
    assert n_chunks % NBUF == 0
    n_groups = n_chunks // NBUF

    mesh = plsc.VectorSubcoreMesh(
        core_axis_name="c", subcore_axis_name="s",
        num_cores=NC, num_subcores=NS)

    @functools.partial(
        pl.kernel,
        out_type=jax.ShapeDtypeStruct((n_total, PADW), jnp.float32),
        mesh=mesh,
        scratch_types=[
            pltpu.VMEM((per_w,), jnp.int32),
            pltpu.VMEM((NBUF, CHUNK, PADW), jnp.float32),
            pltpu.SemaphoreType.DMA((NBUF,)),
            pltpu.SemaphoreType.DMA((NBUF,)),
        ],
        compiler_params=pltpu.CompilerParams(use_tc_tiling_on_sc=True),
    )
    def gather_kernel(vocab_hbm, idx_hbm, out_hbm, idx_v, rows_v, gsem, wsem):
        wid = lax.axis_index("s") * NC + lax.axis_index("c")
        base = wid * per_w
        pltpu.sync_copy(idx_hbm.at[pl.ds(base, per_w)], idx_v)

        def start_gather(k, j):
            pltpu.async_copy(
                vocab_hbm.at[idx_v.at[pl.ds(k * CHUNK, CHUNK)]],
                rows_v.at[j], gsem.at[j])

        def start_write(k, j):
            pltpu.async_copy(
                rows_v.at[j],
                out_hbm.at[pl.ds(base + k * CHUNK, CHUNK)], wsem.at[j])

        def wait_gather(j):
            pltpu.make_async_copy(
                vocab_hbm.at[idx_v.at[pl.ds(0, CHUNK)]],
                rows_v.at[j], gsem.at[j]).wait()

        def wait_write(j):
            pltpu.make_async_copy(
                rows_v.at[j],
                out_hbm.at[pl.ds(base, CHUNK)], wsem.at[j]).wait()

        for j in range(NBUF):
            start_gather(j, j)

        def body(g, carry):
            for j in range(NBUF):
                wait_gather(j)
                start_write(g * NBUF + j, j)
            for j in range(NBUF):
                wait_write(j)
                start_gather((g + 1) * NBUF + j, j)
            return carry

        lax.fori_loop(0, n_groups - 1, body, 0, unroll=False)

        last = (n_groups - 1) * NBUF
        for j in range(NBUF):
            wait_gather(j)
            start_write(last + j, j)
        for j in range(NBUF):
            wait_write(j)

    return gather_kernel


def kernel(s, vocab):
    b, t = s.shape
    dim = vocab.shape[1]
    vocab_p = jnp.pad(vocab, ((0, 0), (0, PADW - dim)))
    idx = s.reshape(-1).astype(jnp.int32)
    out_p = _make_gather(b * t)(vocab_p, idx)
    return out_p.reshape(b, t, PADW)[:, :, :dim]


# speedup vs baseline: 1.2223x; 1.0022x over previous
import functools

import jax
import jax.numpy as jnp
from jax import lax
from jax.experimental import pallas as pl
from jax.experimental.pallas import tpu as pltpu
from jax.experimental.pallas import tpu_sc as plsc

NC = 2
NS = 16
NW = NC * NS

NBUF = 8
CHUNK = 80
PADW = 128


def _make_gather(n_total: int):
    assert n_total % (NW * CHUNK) == 0
    per_w = n_total // NW
    n_chunks = per_w // CHUNK
    assert n_chunks % NBUF == 0
    n_groups = n_chunks // NBUF

    mesh = plsc.VectorSubcoreMesh(
        core_axis_name="c", subcore_axis_name="s",
        num_cores=NC, num_subcores=NS)

    @functools.partial(
        pl.kernel,
        out_type=jax.ShapeDtypeStruct((n_total, PADW), jnp.float32),
        mesh=mesh,
        scratch_types=[
            pltpu.VMEM((per_w,), jnp.int32),
            pltpu.VMEM((NBUF, CHUNK, PADW), jnp.float32),
            pltpu.SemaphoreType.DMA((NBUF,)),
            pltpu.SemaphoreType.DMA((NBUF,)),
        ],
        compiler_params=pltpu.CompilerParams(use_tc_tiling_on_sc=True),
    )
    def gather_kernel(vocab_hbm, idx_hbm, out_hbm, idx_v, rows_v, gsem, wsem):
        wid = lax.axis_index("s") * NC + lax.axis_index("c")
        base = wid * per_w
        pltpu.sync_copy(idx_hbm.at[pl.ds(base, per_w)], idx_v)

        def start_gather(k, j):
            pltpu.async_copy(
                vocab_hbm.at[idx_v.at[pl.ds(k * CHUNK, CHUNK)]],
                rows_v.at[j], gsem.at[j])

        def start_write(k, j):
            pltpu.async_copy(
                rows_v.at[j],
                out_hbm.at[pl.ds(base + k * CHUNK, CHUNK)], wsem.at[j])

        def wait_gather(j):
            pltpu.make_async_copy(
                vocab_hbm.at[idx_v.at[pl.ds(0, CHUNK)]],
                rows_v.at[j], gsem.at[j]).wait()

        def wait_write(j):
            pltpu.make_async_copy(
                rows_v.at[j],
                out_hbm.at[pl.ds(base, CHUNK)], wsem.at[j]).wait()

        for j in range(NBUF):
            start_gather(j, j)

        def body(g, carry):
            for j in range(NBUF):
                wait_gather(j)
                start_write(g * NBUF + j, j)
            for j in range(NBUF):
                wait_write(j)
                start_gather((g + 1) * NBUF + j, j)
            return carry

        lax.fori_loop(0, n_groups - 1, body, 0, unroll=False)

        last = (n_groups - 1) * NBUF
        for j in range(NBUF):
            wait_gather(j)
            start_write(last + j, j)
        for j in range(NBUF):
            wait_write(j)

    return gather_kernel


def kernel(s, vocab):
    b, t = s.shape
    dim = vocab.shape[1]
    vocab_p = jnp.pad(vocab, ((0, 0), (0, PADW - dim)))
    idx = s.reshape(-1).astype(jnp.int32)
    out_p = _make_gather(b * t)(vocab_p, idx)
    return out_p.reshape(b, t, PADW)[:, :, :dim]


# speedup vs baseline: 1.2243x; 1.0017x over previous
import functools

import jax
import jax.numpy as jnp
from jax import lax
from jax.experimental import pallas as pl
from jax.experimental.pallas import tpu as pltpu
from jax.experimental.pallas import tpu_sc as plsc

NC = 2
NS = 16
NW = NC * NS

NBUF = 10
CHUNK = 64
PADW = 128


def _make_gather(n_total: int):
    assert n_total % (NW * CHUNK) == 0
    per_w = n_total // NW
    n_chunks = per_w // CHUNK
    assert n_chunks % NBUF == 0
    n_groups = n_chunks // NBUF

    mesh = plsc.VectorSubcoreMesh(
        core_axis_name="c", subcore_axis_name="s",
        num_cores=NC, num_subcores=NS)

    @functools.partial(
        pl.kernel,
        out_type=jax.ShapeDtypeStruct((n_total, PADW), jnp.float32),
        mesh=mesh,
        scratch_types=[
            pltpu.VMEM((per_w,), jnp.int32),
            pltpu.VMEM((NBUF, CHUNK, PADW), jnp.float32),
            pltpu.SemaphoreType.DMA((NBUF,)),
            pltpu.SemaphoreType.DMA((NBUF,)),
        ],
        compiler_params=pltpu.CompilerParams(use_tc_tiling_on_sc=True),
    )
    def gather_kernel(vocab_hbm, idx_hbm, out_hbm, idx_v, rows_v, gsem, wsem):
        wid = lax.axis_index("s") * NC + lax.axis_index("c")
        base = wid * per_w
        pltpu.sync_copy(idx_hbm.at[pl.ds(base, per_w)], idx_v)

        def start_gather(k, j):
            pltpu.async_copy(
                vocab_hbm.at[idx_v.at[pl.ds(k * CHUNK, CHUNK)]],
                rows_v.at[j], gsem.at[j])

        def start_write(k, j):
            pltpu.async_copy(
                rows_v.at[j],
                out_hbm.at[pl.ds(base + k * CHUNK, CHUNK)], wsem.at[j])

        def wait_gather(j):
            pltpu.make_async_copy(
                vocab_hbm.at[idx_v.at[pl.ds(0, CHUNK)]],
                rows_v.at[j], gsem.at[j]).wait()

        def wait_write(j):
            pltpu.make_async_copy(
                rows_v.at[j],
                out_hbm.at[pl.ds(base, CHUNK)], wsem.at[j]).wait()

        for j in range(NBUF):
            start_gather(j, j)

        def body(g, carry):
            for j in range(NBUF):
                wait_gather(j)
                start_write(g * NBUF + j, j)
            for j in range(NBUF):
                wait_write(j)
                start_gather((g + 1) * NBUF + j, j)
            return carry

        lax.fori_loop(0, n_groups - 1, body, 0, unroll=False)

        last = (n_groups - 1) * NBUF
        for j in range(NBUF):
            wait_gather(j)
            start_write(last + j, j)
        for j in range(NBUF):
            wait_write(j)

    return gather_kernel


def kernel(s, vocab):
    b, t = s.shape
    dim = vocab.shape[1]
    vocab_p = jnp.pad(vocab, ((0, 0), (0, PADW - dim)))
    idx = s.reshape(-1).astype(jnp.int32)
    out_p = _make_gather(b * t)(vocab_p, idx)
    return out_p.reshape(b, t, PADW)[:, :, :dim]
